# baseline (speedup 1.0000x reference)
import functools

import jax
import jax.numpy as jnp
from jax import lax
from jax.experimental import pallas as pl
from jax.experimental.pallas import tpu as pltpu
from jax.experimental.pallas import tpu_sc as plsc

N = 10000
E = 320000
D = 128

NC = 2
NS = 16
NW = NC * NS
EPW = E // NW
CHUNK = 80
NCHUNK = EPW // CHUNK
N_PAD = 10240
RPT = N_PAD // NS
ZROWS = 128
ROWB = 2000

SUB = 80
GSUB = 2
GEDGE = SUB * GSUB
NGRP = 64
TEDGE = GEDGE * NGRP
E_PAD = NW * TEDGE + 2 * GEDGE
NIS = 4

_MESH = plsc.VectorSubcoreMesh(
    core_axis_name="c", subcore_axis_name="s", num_cores=NC, num_subcores=NS)


def _zero_rows(buf, nrows):
    width = buf.shape[1]

    def zrow(i, carry):
        for j in range(width // 16):
            buf[i, pl.ds(j * 16, 16)] = jnp.zeros((16,), jnp.float32)
        return carry

    lax.fori_loop(0, nrows, zrow, 0)


def _sid_dispatch(sid, fn):
    for k in range(NS):
        pl.when(sid == k)(functools.partial(fn, k))


_SC_DEG_SCRATCH = (
    [pltpu.VMEM((SUB,), jnp.int32)] * (NIS * GSUB) +
    [pltpu.VMEM((GEDGE,), jnp.float32)] * NIS +
    [pltpu.VMEM((GEDGE, D), jnp.float32)] * 2 +
    [pltpu.VMEM_SHARED((N_PAD, D), jnp.float32)] +
    [pltpu.SemaphoreType.DMA] * (NIS + 2)
)


def _sc_deg_body(dst_hbm, ew_hbm, out_hbm, *sc):
    o = 0
    dsts = tuple(sc[o + i * GSUB:o + (i + 1) * GSUB] for i in range(NIS))
    o += NIS * GSUB
    ews = sc[o:o + NIS]
    o += NIS
    rows = sc[o:o + 2]
    o += 2
    acc_sp = sc[o]
    o += 1
    isem = sc[o:o + NIS]
    ssem = sc[o + NIS:o + NIS + 2]
    z_v = rows[0].at[pl.ds(0, ZROWS)]

    core = lax.axis_index("c")
    sid = lax.axis_index("s")
    wid = core * NS + sid

    _zero_rows(rows[0], GEDGE)
    _zero_rows(rows[1], GEDGE)

    def zcopy(k):
        for b in range(RPT // ZROWS):
            pltpu.sync_copy(z_v, acc_sp.at[pl.ds(k * RPT + b * ZROWS, ZROWS)])

    _sid_dispatch(sid, zcopy)
    plsc.subcore_barrier()

    def fire_idx(slot, grp):
        base = wid * TEDGE + grp * GEDGE
        ds_ = []
        for b in range(GSUB):
            ds_.append(pltpu.async_copy(dst_hbm.at[pl.ds(base + b * SUB, SUB)],
                                        dsts[slot][b], isem[slot]))
        ds_.append(pltpu.async_copy(ew_hbm.at[pl.ds(base, GEDGE)], ews[slot],
                                    isem[slot]))
        return ds_

    def wait_idx(slot, grp):
        base = wid * TEDGE + grp * GEDGE
        for b in range(GSUB):
            pltpu.make_async_copy(dst_hbm.at[pl.ds(base + b * SUB, SUB)],
                                  dsts[slot][b], isem[slot]).wait()
        pltpu.make_async_copy(ew_hbm.at[pl.ds(base, GEDGE)], ews[slot],
                              isem[slot]).wait()

    def drain_scatters(rslot, islot):
        for b in range(GSUB):
            pltpu.make_async_copy(rows[rslot].at[pl.ds(b * SUB, SUB)],
                                  acc_sp.at[dsts[islot][b]],
                                  ssem[rslot]).wait()

    def process(rslot, islot):
        for b in range(GSUB):
            def row16(q, rc, b=b):
                wv = ews[islot][pl.ds(b * SUB + q * 16, 16)]
                for l in range(16):
                    rows[rslot][b * SUB + q * 16 + l, pl.ds(0, 16)] = (
                        jnp.full((16,), wv[l], jnp.float32))
                return rc

            lax.fori_loop(0, SUB // 16, row16, 0, unroll=5)
            pltpu.async_copy(rows[rslot].at[pl.ds(b * SUB, SUB)],
                             acc_sp.at[dsts[islot][b]], ssem[rslot], add=True)

    for d in fire_idx(0, 0):
        d.wait()
    fire_idx(1, 1)

    def quad(p, carry):
        for u in range(NIS):
            s = NIS * p + u
            rslot, rother = u % 2, 1 - u % 2

            @pl.when(s > 0)
            def _():
                drain_scatters(rother, (u + 3) % NIS)

            wait_idx((u + 1) % NIS, s + 1)
            fire_idx((u + 2) % NIS, s + 2)
            process(rslot, u)
        return carry

    lax.fori_loop(0, NGRP // NIS, quad, 0)
    drain_scatters(1, 3)
    wait_idx(1, NGRP + 1)
    plsc.subcore_barrier()

    def drain(k):
        for b in range(RPT // ZROWS):
            off = k * RPT + b * ZROWS
            pltpu.sync_copy(acc_sp.at[pl.ds(off, ZROWS)], z_v)
            pltpu.sync_copy(z_v, out_hbm.at[core, pl.ds(off, ZROWS)])

    _sid_dispatch(sid, drain)


_SC_AGG_SCRATCH = (
    [pltpu.VMEM((SUB,), jnp.int32)] * (NIS * GSUB) +
    [pltpu.VMEM((SUB,), jnp.int32)] * (NIS * GSUB) +
    [pltpu.VMEM((GEDGE,), jnp.float32)] * NIS +
    [pltpu.VMEM((GEDGE, D), jnp.float32)] * 2 +
    [pltpu.VMEM_SHARED((N_PAD, D), jnp.float32)] +
    [pltpu.SemaphoreType.DMA] * (NIS + 4)
)


def _sc_agg_body(g_hbm, src_hbm, dst_hbm, ew_hbm, out_hbm, *sc):
    o = 0
    srcs = tuple(sc[o + i * GSUB:o + (i + 1) * GSUB] for i in range(NIS))
    o += NIS * GSUB
    dsts = tuple(sc[o + i * GSUB:o + (i + 1) * GSUB] for i in range(NIS))
    o += NIS * GSUB
    ews = sc[o:o + NIS]
    o += NIS
    rows = sc[o:o + 2]
    o += 2
    acc_sp = sc[o]
    o += 1
    isem = sc[o:o + NIS]
    o += NIS
    gsem = sc[o:o + 2]
    ssem = sc[o + 2:o + 4]
    z_v = rows[0].at[pl.ds(0, ZROWS)]

    core = lax.axis_index("c")
    sid = lax.axis_index("s")
    wid = core * NS + sid

    _zero_rows(z_v, ZROWS)

    def zcopy(k):
        for b in range(RPT // ZROWS):
            pltpu.sync_copy(z_v, acc_sp.at[pl.ds(k * RPT + b * ZROWS, ZROWS)])

    _sid_dispatch(sid, zcopy)
    plsc.subcore_barrier()

    def fire_idx(slot, grp):
        base = wid * TEDGE + grp * GEDGE
        ds_ = []
        for b in range(GSUB):
            sl = pl.ds(base + b * SUB, SUB)
            ds_.append(pltpu.async_copy(src_hbm.at[sl], srcs[slot][b],
                                        isem[slot]))
            ds_.append(pltpu.async_copy(dst_hbm.at[sl], dsts[slot][b],
                                        isem[slot]))
        ds_.append(pltpu.async_copy(ew_hbm.at[pl.ds(base, GEDGE)], ews[slot],
                                    isem[slot]))
        return ds_

    def load_idx(slot, grp):
        for d in fire_idx(slot, grp):
            d.wait()

    def wait_idx(slot, grp):
        base = wid * TEDGE + grp * GEDGE
        for b in range(GSUB):
            sl = pl.ds(base + b * SUB, SUB)
            pltpu.make_async_copy(src_hbm.at[sl], srcs[slot][b],
                                  isem[slot]).wait()
            pltpu.make_async_copy(dst_hbm.at[sl], dsts[slot][b],
                                  isem[slot]).wait()
        pltpu.make_async_copy(ew_hbm.at[pl.ds(base, GEDGE)], ews[slot],
                              isem[slot]).wait()

    def fire_gathers(rslot, islot):
        for b in range(GSUB):
            pltpu.async_copy(g_hbm.at[srcs[islot][b]],
                             rows[rslot].at[pl.ds(b * SUB, SUB)], gsem[rslot])

    def drain_scatters(rslot, islot):
        for b in range(GSUB):
            pltpu.make_async_copy(rows[rslot].at[pl.ds(b * SUB, SUB)],
                                  acc_sp.at[dsts[islot][b]],
                                  ssem[rslot]).wait()

    def process(rslot, islot):
        for b in range(GSUB):
            pltpu.make_async_copy(g_hbm.at[srcs[islot][b]],
                                  rows[rslot].at[pl.ds(b * SUB, SUB)],
                                  gsem[rslot]).wait()

            def row16(q, rc, b=b):
                wv = ews[islot][pl.ds(b * SUB + q * 16, 16)]
                for l in range(16):
                    w = jnp.full((16,), wv[l], jnp.float32)
                    r = b * SUB + q * 16 + l
                    for j in range(D // 16):
                        slc = pl.ds(j * 16, 16)
                        rows[rslot][r, slc] = rows[rslot][r, slc] * w
                return rc

            lax.fori_loop(0, SUB // 16, row16, 0, unroll=2)
            pltpu.async_copy(rows[rslot].at[pl.ds(b * SUB, SUB)],
                             acc_sp.at[dsts[islot][b]], ssem[rslot], add=True)

    load_idx(0, 0)
    fire_gathers(0, 0)
    fire_idx(1, 1)

    def quad(p, carry):
        for u in range(NIS):
            s = NIS * p + u
            rslot, rother = u % 2, 1 - u % 2
            iu, inx, inx2 = u, (u + 1) % NIS, (u + 2) % NIS

            @pl.when(s > 0)
            def _():
                drain_scatters(rother, (u + 3) % NIS)

            wait_idx(inx, s + 1)
            fire_gathers(rother, inx)
            fire_idx(inx2, s + 2)
            process(rslot, iu)
        return carry

    lax.fori_loop(0, NGRP // NIS, quad, 0)
    drain_scatters(1, 3)
    for b in range(GSUB):
        pltpu.make_async_copy(g_hbm.at[srcs[0][b]],
                              rows[0].at[pl.ds(b * SUB, SUB)], gsem[0]).wait()
    wait_idx(1, NGRP + 1)
    plsc.subcore_barrier()

    def drain(k):
        for b in range(RPT // ZROWS):
            off = k * RPT + b * ZROWS
            pltpu.sync_copy(acc_sp.at[pl.ds(off, ZROWS)], z_v)
            pltpu.sync_copy(z_v, out_hbm.at[core, pl.ds(off, ZROWS)])

    _sid_dispatch(sid, drain)


_sc_deg = pl.kernel(
    _sc_deg_body,
    out_type=jax.ShapeDtypeStruct((NC, N_PAD, D), jnp.float32),
    mesh=_MESH, scratch_types=_SC_DEG_SCRATCH)

_sc_agg = pl.kernel(
    _sc_agg_body,
    out_type=jax.ShapeDtypeStruct((NC, N_PAD, D), jnp.float32),
    mesh=_MESH, scratch_types=_SC_AGG_SCRATCH)


def _dinv_block(dp_ref):
    deg = dp_ref[0, :, 0:1] + dp_ref[1, :, 0:1] + 1.0
    return lax.rsqrt(deg)


def _tc_pre_body(dp_ref, x_ref, w_ref, o_ref):
    dinv = _dinv_block(dp_ref)
    o_ref[...] = jnp.dot(x_ref[...], w_ref[...],
                         preferred_element_type=jnp.float32) * dinv


def _tc_mid_body(dp_ref, sp_ref, g_ref, b_ref, w_ref, o_ref):
    dinv = _dinv_block(dp_ref)
    s = sp_ref[0] + sp_ref[1]
    pre = (s + g_ref[...]) * dinv + b_ref[...]
    h = pre * jax.nn.sigmoid(pre)
    o_ref[...] = jnp.dot(h, w_ref[...],
                         preferred_element_type=jnp.float32) * dinv


def _tc_fin_body(dp_ref, sp_ref, g_ref, b_ref, o_ref):
    i = pl.program_id(0)
    dinv = _dinv_block(dp_ref)
    s = sp_ref[0] + sp_ref[1]
    pre = (s + g_ref[...]) * dinv + b_ref[...]
    h = pre * jax.nn.sigmoid(pre)
    part = jnp.sum(h, axis=0, keepdims=True) * (1.0 / N)

    @pl.when(i == 0)
    def _():
        o_ref[...] = part

    @pl.when(i > 0)
    def _():
        o_ref[...] = o_ref[...] + part


_DP_SPEC = pl.BlockSpec((NC, ROWB, D), lambda i: (0, i, 0))
_SP_SPEC = pl.BlockSpec((NC, ROWB, D), lambda i: (0, i, 0))
_ROW_SPEC = pl.BlockSpec((ROWB, D), lambda i: (i, 0))
_W_SPEC = pl.BlockSpec((D, D), lambda i: (0, 0))
_B_SPEC = pl.BlockSpec((1, D), lambda i: (0, 0))

_tc_pre = pl.pallas_call(
    _tc_pre_body,
    grid=(N // ROWB,),
    in_specs=[_DP_SPEC, _ROW_SPEC, _W_SPEC],
    out_specs=_ROW_SPEC,
    out_shape=jax.ShapeDtypeStruct((N, D), jnp.float32),
)

_tc_mid = pl.pallas_call(
    _tc_mid_body,
    grid=(N // ROWB,),
    in_specs=[_DP_SPEC, _SP_SPEC, _ROW_SPEC, _B_SPEC, _W_SPEC],
    out_specs=_ROW_SPEC,
    out_shape=jax.ShapeDtypeStruct((N, D), jnp.float32),
)

_tc_fin = pl.pallas_call(
    _tc_fin_body,
    grid=(N // ROWB,),
    in_specs=[_DP_SPEC, _SP_SPEC, _ROW_SPEC, _B_SPEC],
    out_specs=_B_SPEC,
    out_shape=jax.ShapeDtypeStruct((1, D), jnp.float32),
)


def kernel(x, edge_index, edge_weight, W1, b1, W2, b2, W3, b3):
    src = edge_index[0]
    dst = edge_index[1]
    ew = edge_weight

    pad = E_PAD - E
    pad_idx = (jnp.arange(pad, dtype=jnp.int32) * 97) % N
    src_p = jnp.concatenate([src, pad_idx])
    dst_p = jnp.concatenate([dst, pad_idx])
    ew_p = jnp.concatenate([ew, jnp.zeros((pad,), jnp.float32)])

    deg_parts = _sc_deg(dst_p, ew_p)
    g1 = _tc_pre(deg_parts, x, W1)
    s1 = _sc_agg(g1, src_p, dst_p, ew_p)
    g2 = _tc_mid(deg_parts, s1, g1, b1.reshape(1, D), W2)
    s2 = _sc_agg(g2, src_p, dst_p, ew_p)
    g3 = _tc_mid(deg_parts, s2, g2, b2.reshape(1, D), W3)
    s3 = _sc_agg(g3, src_p, dst_p, ew_p)
    return _tc_fin(deg_parts, s3, g3, b3.reshape(1, D))


# speedup vs baseline: 1.1598x; 1.1598x over previous
import functools

import jax
import jax.numpy as jnp
from jax import lax
from jax.experimental import pallas as pl
from jax.experimental.pallas import tpu as pltpu
from jax.experimental.pallas import tpu_sc as plsc

N = 10000
E = 320000
D = 128

NC = 2
NS = 16
NW = NC * NS
EPW = E // NW
CHUNK = 80
NCHUNK = EPW // CHUNK
N_PAD = 10240
RPT = N_PAD // NS
ZROWS = 128
ROWB = 2000

SUB = 80
GSUB = 2
GEDGE = SUB * GSUB
NGRP = 64
TEDGE = GEDGE * NGRP
E_PAD = NW * TEDGE + 2 * GEDGE
NIS = 4

_MESH = plsc.VectorSubcoreMesh(
    core_axis_name="c", subcore_axis_name="s", num_cores=NC, num_subcores=NS)


def _zero_rows(buf, nrows):
    width = buf.shape[1]

    def zrow(i, carry):
        for j in range(width // 16):
            buf[i, pl.ds(j * 16, 16)] = jnp.zeros((16,), jnp.float32)
        return carry

    lax.fori_loop(0, nrows, zrow, 0)


def _sid_dispatch(sid, fn):
    for k in range(NS):
        pl.when(sid == k)(functools.partial(fn, k))


_SC_DEG_SCRATCH = (
    [pltpu.VMEM((SUB,), jnp.int32)] * (NIS * GSUB) +
    [pltpu.VMEM((GEDGE,), jnp.float32)] * NIS +
    [pltpu.VMEM((GEDGE, D), jnp.float32)] * 2 +
    [pltpu.VMEM_SHARED((N_PAD, D), jnp.float32)] +
    [pltpu.SemaphoreType.DMA] * (NIS + 2)
)


def _sc_deg_body(dst_hbm, ew_hbm, out_hbm, *sc):
    o = 0
    dsts = tuple(sc[o + i * GSUB:o + (i + 1) * GSUB] for i in range(NIS))
    o += NIS * GSUB
    ews = sc[o:o + NIS]
    o += NIS
    rows = sc[o:o + 2]
    o += 2
    acc_sp = sc[o]
    o += 1
    isem = sc[o:o + NIS]
    ssem = sc[o + NIS:o + NIS + 2]
    z_v = rows[0].at[pl.ds(0, ZROWS)]

    core = lax.axis_index("c")
    sid = lax.axis_index("s")
    wid = core * NS + sid

    _zero_rows(rows[0], GEDGE)
    _zero_rows(rows[1], GEDGE)

    def zcopy(k):
        for b in range(RPT // ZROWS):
            pltpu.sync_copy(z_v, acc_sp.at[pl.ds(k * RPT + b * ZROWS, ZROWS)])

    _sid_dispatch(sid, zcopy)
    plsc.subcore_barrier()

    def fire_idx(slot, grp):
        base = wid * TEDGE + grp * GEDGE
        ds_ = []
        for b in range(GSUB):
            ds_.append(pltpu.async_copy(dst_hbm.at[pl.ds(base + b * SUB, SUB)],
                                        dsts[slot][b], isem[slot]))
        ds_.append(pltpu.async_copy(ew_hbm.at[pl.ds(base, GEDGE)], ews[slot],
                                    isem[slot]))
        return ds_

    def wait_idx(slot, grp):
        base = wid * TEDGE + grp * GEDGE
        for b in range(GSUB):
            pltpu.make_async_copy(dst_hbm.at[pl.ds(base + b * SUB, SUB)],
                                  dsts[slot][b], isem[slot]).wait()
        pltpu.make_async_copy(ew_hbm.at[pl.ds(base, GEDGE)], ews[slot],
                              isem[slot]).wait()

    def drain_scatters(rslot, islot):
        for b in range(GSUB):
            pltpu.make_async_copy(rows[rslot].at[pl.ds(b * SUB, SUB)],
                                  acc_sp.at[dsts[islot][b]],
                                  ssem[rslot]).wait()

    def process(rslot, islot):
        for b in range(GSUB):
            def row16(q, rc, b=b):
                wv = ews[islot][pl.ds(b * SUB + q * 16, 16)]
                for l in range(16):
                    rows[rslot][b * SUB + q * 16 + l, pl.ds(0, 16)] = (
                        jnp.full((16,), wv[l], jnp.float32))
                return rc

            lax.fori_loop(0, SUB // 16, row16, 0)
            pltpu.async_copy(rows[rslot].at[pl.ds(b * SUB, SUB)],
                             acc_sp.at[dsts[islot][b]], ssem[rslot], add=True)

    for d in fire_idx(0, 0):
        d.wait()
    fire_idx(1, 1)

    def quad(p, carry):
        for u in range(NIS):
            s = NIS * p + u
            rslot, rother = u % 2, 1 - u % 2

            @pl.when(s > 0)
            def _():
                drain_scatters(rother, (u + 3) % NIS)

            wait_idx((u + 1) % NIS, s + 1)
            fire_idx((u + 2) % NIS, s + 2)
            process(rslot, u)
        return carry

    lax.fori_loop(0, NGRP // NIS, quad, 0)
    drain_scatters(1, 3)
    wait_idx(1, NGRP + 1)
    plsc.subcore_barrier()

    def drain(k):
        for b in range(RPT // ZROWS):
            off = k * RPT + b * ZROWS
            pltpu.sync_copy(acc_sp.at[pl.ds(off, ZROWS)], z_v)
            pltpu.sync_copy(z_v, out_hbm.at[core, pl.ds(off, ZROWS)])

    _sid_dispatch(sid, drain)


_SC_AGG_SCRATCH = (
    [pltpu.VMEM((SUB,), jnp.int32)] * (NIS * GSUB) +
    [pltpu.VMEM((SUB,), jnp.int32)] * (NIS * GSUB) +
    [pltpu.VMEM((GEDGE,), jnp.float32)] * NIS +
    [pltpu.VMEM((GEDGE, D), jnp.float32)] * 2 +
    [pltpu.VMEM_SHARED((N_PAD, D), jnp.float32)] +
    [pltpu.SemaphoreType.DMA] * (NIS + 4)
)


def _sc_agg_body(g_hbm, src_hbm, dst_hbm, ew_hbm, out_hbm, *sc):
    o = 0
    srcs = tuple(sc[o + i * GSUB:o + (i + 1) * GSUB] for i in range(NIS))
    o += NIS * GSUB
    dsts = tuple(sc[o + i * GSUB:o + (i + 1) * GSUB] for i in range(NIS))
    o += NIS * GSUB
    ews = sc[o:o + NIS]
    o += NIS
    rows = sc[o:o + 2]
    o += 2
    acc_sp = sc[o]
    o += 1
    isem = sc[o:o + NIS]
    o += NIS
    gsem = sc[o:o + 2]
    ssem = sc[o + 2:o + 4]
    z_v = rows[0].at[pl.ds(0, ZROWS)]

    core = lax.axis_index("c")
    sid = lax.axis_index("s")
    wid = core * NS + sid

    _zero_rows(z_v, ZROWS)

    def zcopy(k):
        for b in range(RPT // ZROWS):
            pltpu.sync_copy(z_v, acc_sp.at[pl.ds(k * RPT + b * ZROWS, ZROWS)])

    _sid_dispatch(sid, zcopy)
    plsc.subcore_barrier()

    def fire_idx(slot, grp):
        base = wid * TEDGE + grp * GEDGE
        ds_ = []
        for b in range(GSUB):
            sl = pl.ds(base + b * SUB, SUB)
            ds_.append(pltpu.async_copy(src_hbm.at[sl], srcs[slot][b],
                                        isem[slot]))
            ds_.append(pltpu.async_copy(dst_hbm.at[sl], dsts[slot][b],
                                        isem[slot]))
        ds_.append(pltpu.async_copy(ew_hbm.at[pl.ds(base, GEDGE)], ews[slot],
                                    isem[slot]))
        return ds_

    def load_idx(slot, grp):
        for d in fire_idx(slot, grp):
            d.wait()

    def wait_idx(slot, grp):
        base = wid * TEDGE + grp * GEDGE
        for b in range(GSUB):
            sl = pl.ds(base + b * SUB, SUB)
            pltpu.make_async_copy(src_hbm.at[sl], srcs[slot][b],
                                  isem[slot]).wait()
            pltpu.make_async_copy(dst_hbm.at[sl], dsts[slot][b],
                                  isem[slot]).wait()
        pltpu.make_async_copy(ew_hbm.at[pl.ds(base, GEDGE)], ews[slot],
                              isem[slot]).wait()

    def fire_gathers(rslot, islot):
        for b in range(GSUB):
            pltpu.async_copy(g_hbm.at[srcs[islot][b]],
                             rows[rslot].at[pl.ds(b * SUB, SUB)], gsem[rslot])

    def drain_scatters(rslot, islot):
        for b in range(GSUB):
            pltpu.make_async_copy(rows[rslot].at[pl.ds(b * SUB, SUB)],
                                  acc_sp.at[dsts[islot][b]],
                                  ssem[rslot]).wait()

    def process(rslot, islot):
        for b in range(GSUB):
            pltpu.make_async_copy(g_hbm.at[srcs[islot][b]],
                                  rows[rslot].at[pl.ds(b * SUB, SUB)],
                                  gsem[rslot]).wait()

            def row16(q, rc, b=b):
                wv = ews[islot][pl.ds(b * SUB + q * 16, 16)]
                for l in range(16):
                    w = jnp.full((16,), wv[l], jnp.float32)
                    r = b * SUB + q * 16 + l
                    for j in range(D // 16):
                        slc = pl.ds(j * 16, 16)
                        rows[rslot][r, slc] = rows[rslot][r, slc] * w
                return rc

            lax.fori_loop(0, SUB // 16, row16, 0)
            pltpu.async_copy(rows[rslot].at[pl.ds(b * SUB, SUB)],
                             acc_sp.at[dsts[islot][b]], ssem[rslot], add=True)

    load_idx(0, 0)
    fire_gathers(0, 0)
    fire_idx(1, 1)

    def quad(p, carry):
        for u in range(NIS):
            s = NIS * p + u
            rslot, rother = u % 2, 1 - u % 2
            iu, inx, inx2 = u, (u + 1) % NIS, (u + 2) % NIS

            @pl.when(s > 0)
            def _():
                drain_scatters(rother, (u + 3) % NIS)

            wait_idx(inx, s + 1)
            fire_gathers(rother, inx)
            fire_idx(inx2, s + 2)
            process(rslot, iu)
        return carry

    lax.fori_loop(0, NGRP // NIS, quad, 0)
    drain_scatters(1, 3)
    for b in range(GSUB):
        pltpu.make_async_copy(g_hbm.at[srcs[0][b]],
                              rows[0].at[pl.ds(b * SUB, SUB)], gsem[0]).wait()
    wait_idx(1, NGRP + 1)
    plsc.subcore_barrier()

    def drain(k):
        for b in range(RPT // ZROWS):
            off = k * RPT + b * ZROWS
            pltpu.sync_copy(acc_sp.at[pl.ds(off, ZROWS)], z_v)
            pltpu.sync_copy(z_v, out_hbm.at[core, pl.ds(off, ZROWS)])

    _sid_dispatch(sid, drain)


_sc_deg = pl.kernel(
    _sc_deg_body,
    out_type=jax.ShapeDtypeStruct((NC, N_PAD, D), jnp.float32),
    mesh=_MESH, scratch_types=_SC_DEG_SCRATCH)

_sc_agg = pl.kernel(
    _sc_agg_body,
    out_type=jax.ShapeDtypeStruct((NC, N_PAD, D), jnp.float32),
    mesh=_MESH, scratch_types=_SC_AGG_SCRATCH)


def _dinv_block(dp_ref):
    deg = dp_ref[0, :, 0:1] + dp_ref[1, :, 0:1] + 1.0
    return lax.rsqrt(deg)


def _tc_pre_body(dp_ref, x_ref, w_ref, o_ref):
    dinv = _dinv_block(dp_ref)
    o_ref[...] = jnp.dot(x_ref[...], w_ref[...],
                         preferred_element_type=jnp.float32) * dinv


def _tc_mid_body(dp_ref, sp_ref, g_ref, b_ref, w_ref, o_ref):
    dinv = _dinv_block(dp_ref)
    s = sp_ref[0] + sp_ref[1]
    pre = (s + g_ref[...]) * dinv + b_ref[...]
    h = pre * jax.nn.sigmoid(pre)
    o_ref[...] = jnp.dot(h, w_ref[...],
                         preferred_element_type=jnp.float32) * dinv


def _tc_fin_body(dp_ref, sp_ref, g_ref, b_ref, o_ref):
    i = pl.program_id(0)
    dinv = _dinv_block(dp_ref)
    s = sp_ref[0] + sp_ref[1]
    pre = (s + g_ref[...]) * dinv + b_ref[...]
    h = pre * jax.nn.sigmoid(pre)
    part = jnp.sum(h, axis=0, keepdims=True) * (1.0 / N)

    @pl.when(i == 0)
    def _():
        o_ref[...] = part

    @pl.when(i > 0)
    def _():
        o_ref[...] = o_ref[...] + part


_DP_SPEC = pl.BlockSpec((NC, ROWB, D), lambda i: (0, i, 0))
_SP_SPEC = pl.BlockSpec((NC, ROWB, D), lambda i: (0, i, 0))
_ROW_SPEC = pl.BlockSpec((ROWB, D), lambda i: (i, 0))
_W_SPEC = pl.BlockSpec((D, D), lambda i: (0, 0))
_B_SPEC = pl.BlockSpec((1, D), lambda i: (0, 0))

_tc_pre = pl.pallas_call(
    _tc_pre_body,
    grid=(N // ROWB,),
    in_specs=[_DP_SPEC, _ROW_SPEC, _W_SPEC],
    out_specs=_ROW_SPEC,
    out_shape=jax.ShapeDtypeStruct((N, D), jnp.float32),
)

_tc_mid = pl.pallas_call(
    _tc_mid_body,
    grid=(N // ROWB,),
    in_specs=[_DP_SPEC, _SP_SPEC, _ROW_SPEC, _B_SPEC, _W_SPEC],
    out_specs=_ROW_SPEC,
    out_shape=jax.ShapeDtypeStruct((N, D), jnp.float32),
)

_tc_fin = pl.pallas_call(
    _tc_fin_body,
    grid=(N // ROWB,),
    in_specs=[_DP_SPEC, _SP_SPEC, _ROW_SPEC, _B_SPEC],
    out_specs=_B_SPEC,
    out_shape=jax.ShapeDtypeStruct((1, D), jnp.float32),
)


def kernel(x, edge_index, edge_weight, W1, b1, W2, b2, W3, b3):
    src = edge_index[0]
    dst = edge_index[1]
    ew = edge_weight

    pad = E_PAD - E
    pad_idx = (jnp.arange(pad, dtype=jnp.int32) * 97) % N
    src_p = jnp.concatenate([src, pad_idx])
    dst_p = jnp.concatenate([dst, pad_idx])
    ew_p = jnp.concatenate([ew, jnp.zeros((pad,), jnp.float32)])

    deg_parts = _sc_deg(dst_p, ew_p)
    g1 = _tc_pre(deg_parts, x, W1)
    s1 = _sc_agg(g1, src_p, dst_p, ew_p)
    g2 = _tc_mid(deg_parts, s1, g1, b1.reshape(1, D), W2)
    s2 = _sc_agg(g2, src_p, dst_p, ew_p)
    g3 = _tc_mid(deg_parts, s2, g2, b2.reshape(1, D), W3)
    s3 = _sc_agg(g3, src_p, dst_p, ew_p)
    return _tc_fin(deg_parts, s3, g3, b3.reshape(1, D))
